# baseline (speedup 1.0000x reference)
import functools

import jax
import jax.numpy as jnp
from jax import lax
from jax.experimental import pallas as pl
from jax.experimental.pallas import tpu as pltpu
from jax.experimental.pallas import tpu_sc as plsc

_B, _C, _T, _H, _W = 8, 192, 32, 28, 28
_NR = _B * _H * _W
_NW = 32
_RPW = _NR // _NW
_K = 4
_NCH = _RPW // _K
_PERM = (31, 7, 4, 29, 16, 19, 2, 5, 30, 3, 22, 6, 18, 10, 11, 15, 20, 8,
         24, 9, 25, 13, 14, 17, 23, 0, 21, 26, 1, 28, 27, 12)


@functools.partial(
    pl.kernel,
    mesh=plsc.VectorSubcoreMesh(core_axis_name="c", subcore_axis_name="s"),
    out_type=jax.ShapeDtypeStruct((_NR, _T, _C), jnp.float32),
    scratch_types=[
        pltpu.VMEM((_K, _T, _C), jnp.float32),
        pltpu.SemaphoreType.DMA,
    ],
)
def _shuffle_native(x_hbm, out_hbm, buf, sem):
    wid = lax.axis_index("s") * 2 + lax.axis_index("c")
    base = wid * _RPW

    def chunk(i, _):
        off = base + i * _K
        cps = [
            pltpu.async_copy(
                x_hbm.at[pl.ds(off, _K), _PERM[t], :], buf.at[:, t, :], sem
            )
            for t in range(_T)
        ]
        for cp in cps:
            cp.wait()
        pltpu.sync_copy(buf, out_hbm.at[pl.ds(off, _K)])
        return ()

    lax.fori_loop(0, _NCH, chunk, (), unroll=False)


def kernel(x):
    xt = jnp.transpose(x, (0, 3, 4, 2, 1)).reshape(_NR, _T, _C)
    out3 = _shuffle_native(xt)
    return jnp.transpose(out3.reshape(_B, _H, _W, _T, _C), (0, 4, 3, 1, 2))


# speedup vs baseline: 1.1848x; 1.1848x over previous
import functools

import jax
import jax.numpy as jnp
from jax import lax
from jax.experimental import pallas as pl
from jax.experimental.pallas import tpu as pltpu
from jax.experimental.pallas import tpu_sc as plsc

_B, _C, _T, _H, _W = 8, 192, 32, 28, 28
_NR = _B * _H * _W
_NW = 32
_RPW = _NR // _NW
_K = 7
_NCH = _RPW // _K
_PERM = (31, 7, 4, 29, 16, 19, 2, 5, 30, 3, 22, 6, 18, 10, 11, 15, 20, 8,
         24, 9, 25, 13, 14, 17, 23, 0, 21, 26, 1, 28, 27, 12)


@functools.partial(
    pl.kernel,
    mesh=plsc.VectorSubcoreMesh(core_axis_name="c", subcore_axis_name="s"),
    out_type=jax.ShapeDtypeStruct((_NR, _T, _C), jnp.float32),
    scratch_types=[
        pltpu.VMEM((_K, _T, _C), jnp.float32),
        pltpu.VMEM((_K, _T, _C), jnp.float32),
        pltpu.SemaphoreType.DMA,
        pltpu.SemaphoreType.DMA,
        pltpu.SemaphoreType.DMA,
        pltpu.SemaphoreType.DMA,
    ],
)
def _shuffle_native(x_hbm, out_hbm, buf0, buf1, g0, g1, w0, w1):
    wid = lax.axis_index("s") * 2 + lax.axis_index("c")
    base = wid * _RPW

    def reads(off, buf, sem):
        return [
            pltpu.async_copy(
                x_hbm.at[pl.ds(off, _K), _PERM[t], :], buf.at[:, t, :], sem
            )
            for t in range(_T)
        ]

    def pair(j, _):
        off0 = base + (2 * j) * _K
        off1 = off0 + _K
        r0 = reads(off0, buf0, g0)
        r1 = reads(off1, buf1, g1)
        for cp in r0:
            cp.wait()
        cw0 = pltpu.async_copy(buf0, out_hbm.at[pl.ds(off0, _K)], w0)
        for cp in r1:
            cp.wait()
        cw1 = pltpu.async_copy(buf1, out_hbm.at[pl.ds(off1, _K)], w1)
        cw0.wait()
        cw1.wait()
        return ()

    lax.fori_loop(0, _NCH // 2, pair, (), unroll=False)


def kernel(x):
    xt = jnp.transpose(x, (0, 3, 4, 2, 1)).reshape(_NR, _T, _C)
    out3 = _shuffle_native(xt)
    return jnp.transpose(out3.reshape(_B, _H, _W, _T, _C), (0, 4, 3, 1, 2))


# speedup vs baseline: 1.1948x; 1.0085x over previous
import functools

import jax
import jax.numpy as jnp
from jax import lax
from jax.experimental import pallas as pl
from jax.experimental.pallas import tpu as pltpu
from jax.experimental.pallas import tpu_sc as plsc

_B, _C, _T, _H, _W = 8, 192, 32, 28, 28
_NR = _B * _H * _W
_NW = 32
_RPW = _NR // _NW
_K = 7
_NCH = _RPW // _K
_PERM = (31, 7, 4, 29, 16, 19, 2, 5, 30, 3, 22, 6, 18, 10, 11, 15, 20, 8,
         24, 9, 25, 13, 14, 17, 23, 0, 21, 26, 1, 28, 27, 12)


@functools.partial(
    pl.kernel,
    mesh=plsc.VectorSubcoreMesh(core_axis_name="c", subcore_axis_name="s"),
    out_type=jax.ShapeDtypeStruct((_NR, _T, _C), jnp.float32),
    scratch_types=[
        pltpu.VMEM((_K, _T, _C), jnp.float32),
        pltpu.VMEM((_K, _T, _C), jnp.float32),
        pltpu.SemaphoreType.DMA,
        pltpu.SemaphoreType.DMA,
        pltpu.SemaphoreType.DMA,
        pltpu.SemaphoreType.DMA,
    ],
)
def _shuffle_native(x_hbm, out_hbm, buf0, buf1, g0, g1, w0, w1):
    wid = lax.axis_index("s") * 2 + lax.axis_index("c")
    base = wid * _RPW

    def reads(off, buf, sem):
        return [
            pltpu.async_copy(
                x_hbm.at[pl.ds(off, _K), _PERM[t], :], buf.at[:, t, :], sem
            )
            for t in range(_T)
        ]

    def drain(buf, sem):
        pltpu.make_async_copy(x_hbm.at[pl.ds(base, _K)], buf, sem).wait()

    def pair(j, _):
        off0 = base + (2 * j) * _K
        off1 = off0 + _K

        @pl.when(j > 0)
        def _():
            drain(buf0, w0)
            drain(buf1, w1)

        r0 = reads(off0, buf0, g0)
        r1 = reads(off1, buf1, g1)
        for cp in r0:
            cp.wait()
        pltpu.async_copy(buf0, out_hbm.at[pl.ds(off0, _K)], w0)
        for cp in r1:
            cp.wait()
        pltpu.async_copy(buf1, out_hbm.at[pl.ds(off1, _K)], w1)
        return ()

    lax.fori_loop(0, _NCH // 2, pair, (), unroll=False)
    drain(buf0, w0)
    drain(buf1, w1)


def kernel(x):
    xt = jnp.transpose(x, (0, 3, 4, 2, 1)).reshape(_NR, _T, _C)
    out3 = _shuffle_native(xt)
    return jnp.transpose(out3.reshape(_B, _H, _W, _T, _C), (0, 4, 3, 1, 2))
